# baseline (speedup 1.0000x reference)
import math

import jax
import jax.numpy as jnp
from jax.experimental import pallas as pl
from jax.experimental.pallas import tpu as pltpu

S = 64.0
M = 0.5
EPS = 1e-07
COS_M = math.cos(M)
SIN_M = math.sin(M)

BR = 256
CW = 1280


def _row_loss_body(lab_ref, wf_ref, out_ref):
    br, c = wf_ref.shape
    lab = lab_ref[...]
    acc_exp = jnp.zeros((br, 1), jnp.float32)
    acc_tgt = jnp.zeros((br, 1), jnp.float32)
    start = 0
    while start < c:
        w = min(CW, c - start)
        blk = wf_ref[:, start:start + w]
        cols = jax.lax.broadcasted_iota(jnp.int32, (br, w), 1) + start
        hit = cols == lab
        acc_tgt = acc_tgt + jnp.sum(jnp.where(hit, blk, 0.0), axis=1,
                                    keepdims=True)
        e = jnp.exp(S * blk)
        if start + w >= c and w % 128 != 0:
            e = jnp.where(cols < c, e, 0.0)
        acc_exp = acc_exp + jnp.sum(e, axis=1, keepdims=True)
        start += w
    t_raw = acc_tgt
    t = jnp.clip(t_raw, -1.0 + EPS, 1.0 - EPS)
    num = S * (t * COS_M - jnp.sqrt(1.0 - t * t) * SIN_M)
    den = jnp.exp(num) + (acc_exp - jnp.exp(S * t_raw))
    out_ref[...] = num - jnp.log(den)


def _mean_body(l_ref, o_ref):
    n = l_ref.shape[0] * l_ref.shape[1]
    o_ref[0, 0] = jnp.sum(l_ref[...]) * (-1.0 / n)


def kernel(wf, labels):
    b, c = wf.shape
    lab2 = labels.astype(jnp.int32).reshape(b, 1)
    row_loss = pl.pallas_call(
        _row_loss_body,
        grid=(b // BR,),
        in_specs=[
            pl.BlockSpec((BR, 1), lambda i: (i, 0)),
            pl.BlockSpec((BR, c), lambda i: (i, 0)),
        ],
        out_specs=pl.BlockSpec((BR, 1), lambda i: (i, 0)),
        out_shape=jax.ShapeDtypeStruct((b, 1), jnp.float32),
        compiler_params=pltpu.CompilerParams(
            dimension_semantics=("parallel",),
            vmem_limit_bytes=50 * 1024 * 1024,
        ),
        name="arcface_row_loss",
    )(lab2, wf)
    out = pl.pallas_call(
        _mean_body,
        in_specs=[pl.BlockSpec(memory_space=pltpu.VMEM)],
        out_specs=pl.BlockSpec(memory_space=pltpu.SMEM),
        out_shape=jax.ShapeDtypeStruct((1, 1), jnp.float32),
        name="arcface_mean",
    )(row_loss.reshape(b // 128, 128))
    return out.reshape(())


# speedup vs baseline: 1.0145x; 1.0145x over previous
import math

import jax
import jax.numpy as jnp
from jax.experimental import pallas as pl
from jax.experimental.pallas import tpu as pltpu

S = 64.0
M = 0.5
EPS = 1e-07
COS_M = math.cos(M)
SIN_M = math.sin(M)
EXPC = S * math.log2(math.e)

BR = 128
NSUB = 4


def _row_loss_body(lab_ref, *refs):
    wf_refs, out_ref = refs[:NSUB], refs[NSUB]
    c = wf_refs[0].shape[1]
    nfull = c // 128
    rem = c - nfull * 128
    iota128 = jax.lax.broadcasted_iota(jnp.int32, (BR, 128), 1)
    for k in range(NSUB):
        wfr = wf_refs[k]
        lab = lab_ref[k * BR:(k + 1) * BR, :]
        lab_b = jnp.broadcast_to(lab, (BR, 128))
        acc_e = jnp.zeros((BR, 128), jnp.float32)
        acc_t = jnp.zeros((BR, 128), jnp.float32)
        for j in range(nfull):
            blk = wfr[:, j * 128:(j + 1) * 128]
            acc_e = acc_e + jnp.exp2(blk * EXPC)
            hit = (iota128 + j * 128) == lab_b
            acc_t = acc_t + jnp.where(hit, blk, 0.0)
        rowsum = jnp.sum(acc_e, axis=1, keepdims=True)
        tgt = jnp.sum(acc_t, axis=1, keepdims=True)
        if rem:
            tail = wfr[:, nfull * 128:c]
            cols = jax.lax.broadcasted_iota(jnp.int32, (BR, rem), 1) \
                + nfull * 128
            e_tail = jnp.where(cols < c, jnp.exp2(tail * EXPC), 0.0)
            rowsum = rowsum + jnp.sum(e_tail, axis=1, keepdims=True)
            tgt = tgt + jnp.sum(jnp.where(cols == lab, tail, 0.0),
                                axis=1, keepdims=True)
        t = jnp.clip(tgt, -1.0 + EPS, 1.0 - EPS)
        num = S * (t * COS_M - jnp.sqrt(1.0 - t * t) * SIN_M)
        den = jnp.exp(num) + (rowsum - jnp.exp(S * tgt))
        out_ref[k * BR:(k + 1) * BR, :] = num - jnp.log(den)


def _mean_body(l_ref, o_ref):
    n = l_ref.shape[0] * l_ref.shape[1]
    o_ref[0, 0] = jnp.sum(l_ref[...]) * (-1.0 / n)


def kernel(wf, labels):
    b, c = wf.shape
    step = BR * NSUB
    lab2 = labels.astype(jnp.int32).reshape(b, 1)
    wf_specs = [
        pl.BlockSpec((BR, c), lambda i, k=k: (i * NSUB + k, 0))
        for k in range(NSUB)
    ]
    row_loss = pl.pallas_call(
        _row_loss_body,
        grid=(b // step,),
        in_specs=[pl.BlockSpec((step, 1), lambda i: (i, 0))] + wf_specs,
        out_specs=pl.BlockSpec((step, 1), lambda i: (i, 0)),
        out_shape=jax.ShapeDtypeStruct((b, 1), jnp.float32),
        compiler_params=pltpu.CompilerParams(
            dimension_semantics=("arbitrary",),
            vmem_limit_bytes=56 * 1024 * 1024,
        ),
        name="arcface_row_loss",
    )(lab2, *([wf] * NSUB))
    out = pl.pallas_call(
        _mean_body,
        in_specs=[pl.BlockSpec(memory_space=pltpu.VMEM)],
        out_specs=pl.BlockSpec(memory_space=pltpu.SMEM),
        out_shape=jax.ShapeDtypeStruct((1, 1), jnp.float32),
        name="arcface_mean",
    )(row_loss.reshape(b // 128, 128))
    return out.reshape(())


# speedup vs baseline: 3.9203x; 3.8644x over previous
import math

import jax
import jax.numpy as jnp
from jax.experimental import pallas as pl
from jax.experimental.pallas import tpu as pltpu

S = 64.0
M = 0.5
EPS = 1e-07
COS_M = math.cos(M)
SIN_M = math.sin(M)
EXPC = S * math.log2(math.e)

CB = 80
NSUB = 5


def _loss_body(lab_ref, *refs):
    wf_refs = refs[:NSUB]
    o_ref, acc_e_ref, acc_t_ref = refs[NSUB], refs[NSUB + 1], refs[NSUB + 2]
    i = pl.program_id(0)
    ngrid = pl.num_programs(0)

    @pl.when(i == 0)
    def _():
        acc_e_ref[...] = jnp.zeros_like(acc_e_ref)
        acc_t_ref[...] = jnp.zeros_like(acc_t_ref)

    nb = wf_refs[0].shape[1]
    lab = lab_ref[...]
    iota8 = jax.lax.broadcasted_iota(jnp.int32, (8, nb), 0)
    base_i = i * (NSUB * CB)
    loc_e = jnp.zeros((8, nb), jnp.float32)
    loc_t = jnp.zeros((8, nb), jnp.float32)
    for k in range(NSUB):
        wfr = wf_refs[k]
        lab_rel = lab - (base_i + k * CB)
        for r in range(CB // 8):
            blk = wfr[r * 8:(r + 1) * 8, :]
            hit = (iota8 + r * 8) == lab_rel
            loc_e = loc_e + jnp.exp2(blk * EXPC)
            loc_t = loc_t + jnp.where(hit, blk, 0.0)
    acc_e_ref[...] = acc_e_ref[...] + loc_e
    acc_t_ref[...] = acc_t_ref[...] + loc_t

    @pl.when(i == ngrid - 1)
    def _():
        rowsum = jnp.sum(acc_e_ref[...], axis=0, keepdims=True)
        tgt = jnp.sum(acc_t_ref[...], axis=0, keepdims=True)
        t = jnp.clip(tgt, -1.0 + EPS, 1.0 - EPS)
        num = S * (t * COS_M - jnp.sqrt(1.0 - t * t) * SIN_M)
        den = jnp.exp(num) + (rowsum - jnp.exp(S * tgt))
        loss = num - jnp.log(den)
        o_ref[0, 0] = jnp.sum(loss) * (-1.0 / nb)


def kernel(wf, labels):
    b, c = wf.shape
    wft = wf.T
    lab2 = labels.astype(jnp.int32).reshape(1, b)
    step = NSUB * CB
    wf_specs = [
        pl.BlockSpec((CB, b), lambda i, k=k: (i * NSUB + k, 0))
        for k in range(NSUB)
    ]
    out = pl.pallas_call(
        _loss_body,
        grid=(c // step,),
        in_specs=[pl.BlockSpec((1, b), lambda i: (0, 0))] + wf_specs,
        out_specs=pl.BlockSpec(memory_space=pltpu.SMEM),
        out_shape=jax.ShapeDtypeStruct((1, 1), jnp.float32),
        scratch_shapes=[
            pltpu.VMEM((8, b), jnp.float32),
            pltpu.VMEM((8, b), jnp.float32),
        ],
        compiler_params=pltpu.CompilerParams(
            dimension_semantics=("arbitrary",),
            vmem_limit_bytes=56 * 1024 * 1024,
        ),
        name="arcface_loss",
    )(lab2, *([wft] * NSUB))
    return out.reshape(())
